# baseline (speedup 1.0000x reference)
import functools

import jax
import jax.numpy as jnp
from jax import lax
from jax.experimental import pallas as pl
from jax.experimental.pallas import tpu as pltpu
from jax.experimental.pallas import tpu_sc as plsc

BITVEC = 256
TOK = 16
NTOK = BITVEC // TOK
D = 128
BATCH = 16384
NC, NS = 2, 16
NW = NC * NS
B_PER_W = BATCH // NW
CH = 8
NCHUNK = B_PER_W // CH
TPC = CH * NTOK
NBUF = 4

_mesh = plsc.VectorSubcoreMesh(
    core_axis_name="c", subcore_axis_name="s", num_cores=NC, num_subcores=NS)


@functools.partial(
    pl.kernel,
    out_type=jax.ShapeDtypeStruct((BATCH, NTOK * D), jnp.float32),
    mesh=_mesh,
    compiler_params=pltpu.CompilerParams(
        needs_layout_passes=False, use_tc_tiling_on_sc=True),
    scratch_types=[
        [pltpu.VMEM((CH, BITVEC), jnp.int32)] * NBUF,
        [pltpu.VMEM((TPC,), jnp.int32)] * NBUF,
        [pltpu.VMEM((TPC, D), jnp.float32)] * NBUF,
        [pltpu.SemaphoreType.DMA] * NBUF,
        [pltpu.SemaphoreType.DMA] * NBUF,
        [pltpu.SemaphoreType.DMA] * NBUF,
    ],
)
def _emb_kernel(x_hbm, w_hbm, out_hbm, xv, idxv, rowsv, sin, sg, sout):
    wid = lax.axis_index("s") * NC + lax.axis_index("c")
    row0 = wid * B_PER_W
    lane = lax.iota(jnp.int32, 16)
    pow2 = jnp.left_shift(jnp.int32(1), lane)

    def start_in(i, b):
        return pltpu.async_copy(
            x_hbm.at[pl.ds(row0 + i * CH, CH), :], xv[b], sin[b])

    def start_out(i, b):
        return pltpu.async_copy(
            rowsv[b].reshape(CH, NTOK * D),
            out_hbm.at[pl.ds(row0 + i * CH, CH), :], sout[b])

    def pack(b):
        xb = xv[b]
        for r in range(CH):
            vals = jnp.zeros((16,), jnp.int32)
            for t in range(NTOK // 2):
                v_lo = xb[r, pl.ds(t * TOK, TOK)]
                v_hi = xb[r, pl.ds((t + 8) * TOK, TOK)]
                comb = v_lo + jnp.left_shift(v_hi, 16)
                s_tok = jnp.sum(comb * pow2)
                lo = jnp.bitwise_and(s_tok, 0xFFFF)
                hi = lax.shift_right_logical(s_tok, 16)
                vals = jnp.where(lane == t, lo, vals)
                vals = jnp.where(lane == t + 8, hi, vals)
            idxv[b][pl.ds(r * NTOK, NTOK)] = vals

    for p in range(3):
        start_in(p, p)

    def ring_body(j, carry):
        for b in range(NBUF):
            i = j * NBUF + b
            pltpu.make_async_copy(
                x_hbm.at[pl.ds(row0 + i * CH, CH), :], xv[b], sin[b]).wait()
            pack(b)
            @pl.when(i >= NBUF)
            def _():
                pltpu.make_async_copy(
                    rowsv[b].reshape(CH, NTOK * D),
                    out_hbm.at[pl.ds(row0 + i * CH, CH), :], sout[b]).wait()
            pltpu.async_copy(w_hbm.at[idxv[b]], rowsv[b], sg[b])
            @pl.when(i + 3 < NCHUNK)
            def _():
                start_in(i + 3, (b + 3) % NBUF)
            pb = (b - 1) % NBUF
            @pl.when(i >= 1)
            def _():
                pltpu.make_async_copy(
                    w_hbm.at[idxv[pb]], rowsv[pb], sg[pb]).wait()
                start_out(i - 1, pb)
        return carry

    lax.fori_loop(0, NCHUNK // NBUF, ring_body, 0)
    lb = (NCHUNK - 1) % NBUF
    pltpu.make_async_copy(w_hbm.at[idxv[lb]], rowsv[lb], sg[lb]).wait()
    start_out(NCHUNK - 1, lb)
    for b in range(NBUF):
        pltpu.make_async_copy(
            rowsv[b].reshape(CH, NTOK * D),
            out_hbm.at[pl.ds(row0, CH), :], sout[b]).wait()


def kernel(x, W):
    return _emb_kernel(x, W)


# speedup vs baseline: 1.0101x; 1.0101x over previous
import functools

import jax
import jax.numpy as jnp
from jax import lax
from jax.experimental import pallas as pl
from jax.experimental.pallas import tpu as pltpu
from jax.experimental.pallas import tpu_sc as plsc

BITVEC = 256
TOK = 16
NTOK = BITVEC // TOK
D = 128
BATCH = 16384
NC, NS = 2, 16
NW = NC * NS
B_PER_W = BATCH // NW
CH = 8
NCHUNK = B_PER_W // CH
TPC = CH * NTOK
NBUF = 4

_mesh = plsc.VectorSubcoreMesh(
    core_axis_name="c", subcore_axis_name="s", num_cores=NC, num_subcores=NS)


@functools.partial(
    pl.kernel,
    out_type=jax.ShapeDtypeStruct((BATCH, NTOK * D), jnp.float32),
    mesh=_mesh,
    compiler_params=pltpu.CompilerParams(
        needs_layout_passes=False, use_tc_tiling_on_sc=True),
    scratch_types=[
        [pltpu.VMEM((CH, BITVEC), jnp.int32)] * NBUF,
        [pltpu.VMEM((TPC,), jnp.int32)] * NBUF,
        [pltpu.VMEM((TPC, D), jnp.float32)] * NBUF,
        [pltpu.SemaphoreType.DMA] * NBUF,
        [pltpu.SemaphoreType.DMA] * NBUF,
        [pltpu.SemaphoreType.DMA] * NBUF,
    ],
)
def _emb_kernel(x_hbm, w_hbm, out_hbm, xv, idxv, rowsv, sin, sg, sout):
    wid = lax.axis_index("s") * NC + lax.axis_index("c")
    row0 = wid * B_PER_W
    lane = lax.iota(jnp.int32, 16)
    pow2 = jnp.left_shift(jnp.int32(1), lane)

    def start_in(i, b):
        return pltpu.async_copy(
            x_hbm.at[pl.ds(row0 + i * CH, CH), :], xv[b], sin[b])

    def start_out(i, b):
        return pltpu.async_copy(
            rowsv[b].reshape(CH, NTOK * D),
            out_hbm.at[pl.ds(row0 + i * CH, CH), :], sout[b])

    def pack(b):
        xb = xv[b]
        for r in range(CH):
            vals = jnp.zeros((16,), jnp.int32)
            for t in range(NTOK):
                v = xb[r, pl.ds(t * TOK, TOK)]
                s_tok = jnp.sum(v * pow2)
                vals = jnp.where(lane == t, s_tok, vals)
            idxv[b][pl.ds(r * NTOK, NTOK)] = vals

    for p in range(3):
        start_in(p, p)

    def ring_body(j, carry):
        for b in range(NBUF):
            i = j * NBUF + b
            pltpu.make_async_copy(
                x_hbm.at[pl.ds(row0 + i * CH, CH), :], xv[b], sin[b]).wait()
            pack(b)
            @pl.when(i >= NBUF)
            def _():
                pltpu.make_async_copy(
                    rowsv[b].reshape(CH, NTOK * D),
                    out_hbm.at[pl.ds(row0 + i * CH, CH), :], sout[b]).wait()
            pltpu.async_copy(w_hbm.at[idxv[b]], rowsv[b], sg[b])
            @pl.when(i + 3 < NCHUNK)
            def _():
                start_in(i + 3, (b + 3) % NBUF)
            pb = (b - 1) % NBUF
            @pl.when(i >= 1)
            def _():
                pltpu.make_async_copy(
                    w_hbm.at[idxv[pb]], rowsv[pb], sg[pb]).wait()
                start_out(i - 1, pb)
        return carry

    lax.fori_loop(0, NCHUNK // NBUF, ring_body, 0)
    lb = (NCHUNK - 1) % NBUF
    pltpu.make_async_copy(w_hbm.at[idxv[lb]], rowsv[lb], sg[lb]).wait()
    start_out(NCHUNK - 1, lb)
    for b in range(NBUF):
        pltpu.make_async_copy(
            rowsv[b].reshape(CH, NTOK * D),
            out_hbm.at[pl.ds(row0, CH), :], sout[b]).wait()


def kernel(x, W):
    return _emb_kernel(x, W)
